# baseline (speedup 1.0000x reference)
import functools

import jax
import jax.numpy as jnp
from jax import lax
from jax.experimental import pallas as pl
from jax.experimental.pallas import tpu as pltpu
from jax.experimental.pallas import tpu_sc as plsc

_NCYCLE = 12
_EMBED = 128
_BATCH = 16384
_NC = 2
_NS = 16
_NW = _NC * _NS
_BPW = _BATCH // _NW
_SLAB = 128
_NSLAB = _BPW // _SLAB
_WSLAB = 128
_NWSLAB = _BPW // _WSLAB
_LANES = 16


@functools.partial(
    pl.kernel,
    mesh=plsc.VectorSubcoreMesh(core_axis_name="c", subcore_axis_name="s"),
    out_type=jax.ShapeDtypeStruct((_BATCH, _EMBED), jnp.float32),
    scratch_types=[
        pltpu.VMEM((_BPW,), jnp.float32),
        pltpu.VMEM_SHARED((_NS * _NCYCLE, _EMBED), jnp.float32),
        pltpu.VMEM((_NSLAB, _SLAB), jnp.int32),
        pltpu.VMEM((_BPW, _EMBED), jnp.float32),
        pltpu.SemaphoreType.DMA,
        pltpu.SemaphoreType.DMA,
        pltpu.SemaphoreType.DMA,
    ],
)
def _seasonal_embed(t_hbm, w_hbm, out_hbm, t_v, w_sh, idx_v, rows_v, tsem,
                    gsem, wsem):
    sid = lax.axis_index("s")
    wid = sid * _NC + lax.axis_index("c")
    base = wid * _BPW

    t_cp = pltpu.async_copy(t_hbm.at[pl.ds(base, _BPW)], t_v, tsem)

    @pl.when(sid == 0)
    def _():
        w_cps = [
            pltpu.async_copy(
                w_hbm, w_sh.at[pl.ds(k * _NCYCLE, _NCYCLE)], gsem
            )
            for k in range(_NS)
        ]
        for cp in w_cps:
            cp.wait()

    bias = (sid * _NCYCLE).astype(jnp.int32)
    t_cp.wait()
    for c in range(_SLAB // _LANES):
        x = t_v[pl.ds(c * _LANES, _LANES)] * jnp.float32(_NCYCLE)
        x = lax.rem(x, jnp.float32(_NCYCLE))
        idx_v[0, pl.ds(c * _LANES, _LANES)] = x.astype(jnp.int32) + bias

    plsc.subcore_barrier()

    gathers = [
        pltpu.async_copy(
            w_sh.at[idx_v.at[0]], rows_v.at[pl.ds(0, _SLAB)], gsem
        )
    ]
    for j in range(1, _NSLAB):
        for c in range(_SLAB // _LANES):
            x = t_v[pl.ds(j * _SLAB + c * _LANES, _LANES)] * jnp.float32(
                _NCYCLE)
            x = lax.rem(x, jnp.float32(_NCYCLE))
            idx_v[j, pl.ds(c * _LANES, _LANES)] = x.astype(jnp.int32) + bias
        gathers.append(
            pltpu.async_copy(
                w_sh.at[idx_v.at[j]], rows_v.at[pl.ds(j * _SLAB, _SLAB)], gsem
            )
        )
    writes = []
    per_write = _WSLAB // _SLAB
    for k in range(_NWSLAB):
        for j in range(k * per_write, (k + 1) * per_write):
            gathers[j].wait()
        writes.append(
            pltpu.async_copy(
                rows_v.at[pl.ds(k * _WSLAB, _WSLAB)],
                out_hbm.at[pl.ds(base + k * _WSLAB, _WSLAB)],
                wsem,
            )
        )
    for cp in writes:
        cp.wait()


def kernel(t, W):
    return _seasonal_embed(t, W)


# speedup vs baseline: 1.0427x; 1.0427x over previous
import functools

import jax
import jax.numpy as jnp
from jax import lax
from jax.experimental import pallas as pl
from jax.experimental.pallas import tpu as pltpu
from jax.experimental.pallas import tpu_sc as plsc

_NCYCLE = 12
_EMBED = 128
_BATCH = 16384
_NC = 2
_NS = 16
_NW = _NC * _NS
_BPW = _BATCH // _NW
_SLAB = 128
_NSLAB = _BPW // _SLAB
_WSLAB = 128
_NWSLAB = _BPW // _WSLAB
_LANES = 16


@functools.partial(
    pl.kernel,
    mesh=plsc.VectorSubcoreMesh(core_axis_name="c", subcore_axis_name="s"),
    out_type=jax.ShapeDtypeStruct((_BATCH, _EMBED), jnp.float32),
    scratch_types=[
        pltpu.VMEM((_BPW,), jnp.float32),
        pltpu.VMEM_SHARED((_NCYCLE, _EMBED), jnp.float32),
        pltpu.VMEM((_NSLAB, _SLAB), jnp.int32),
        pltpu.VMEM((_BPW, _EMBED), jnp.float32),
        pltpu.SemaphoreType.DMA,
        pltpu.SemaphoreType.DMA,
        pltpu.SemaphoreType.DMA,
    ],
)
def _seasonal_embed(t_hbm, w_hbm, out_hbm, t_v, w_sh, idx_v, rows_v, tsem,
                    gsem, wsem):
    sid = lax.axis_index("s")
    wid = sid * _NC + lax.axis_index("c")
    base = wid * _BPW

    t_cp = pltpu.async_copy(t_hbm.at[pl.ds(base, _BPW)], t_v, tsem)

    @pl.when(sid == 0)
    def _():
        pltpu.sync_copy(w_hbm, w_sh)

    t_cp.wait()
    for c in range(_SLAB // _LANES):
        x = t_v[pl.ds(c * _LANES, _LANES)] * jnp.float32(_NCYCLE)
        x = lax.rem(x, jnp.float32(_NCYCLE))
        idx_v[0, pl.ds(c * _LANES, _LANES)] = x.astype(jnp.int32)

    plsc.subcore_barrier()

    gathers = [
        pltpu.async_copy(
            w_sh.at[idx_v.at[0]], rows_v.at[pl.ds(0, _SLAB)], gsem
        )
    ]
    for j in range(1, _NSLAB):
        for c in range(_SLAB // _LANES):
            x = t_v[pl.ds(j * _SLAB + c * _LANES, _LANES)] * jnp.float32(
                _NCYCLE)
            x = lax.rem(x, jnp.float32(_NCYCLE))
            idx_v[j, pl.ds(c * _LANES, _LANES)] = x.astype(jnp.int32)
        gathers.append(
            pltpu.async_copy(
                w_sh.at[idx_v.at[j]], rows_v.at[pl.ds(j * _SLAB, _SLAB)], gsem
            )
        )
    writes = []
    per_write = _WSLAB // _SLAB
    for k in range(_NWSLAB):
        for j in range(k * per_write, (k + 1) * per_write):
            gathers[j].wait()
        writes.append(
            pltpu.async_copy(
                rows_v.at[pl.ds(k * _WSLAB, _WSLAB)],
                out_hbm.at[pl.ds(base + k * _WSLAB, _WSLAB)],
                wsem,
            )
        )
    for cp in writes:
        cp.wait()


def kernel(t, W):
    return _seasonal_embed(t, W)
